# baseline (speedup 1.0000x reference)
import functools

import jax
import jax.numpy as jnp
from jax import lax
from jax.experimental import pallas as pl
from jax.experimental.pallas import tpu as pltpu
from jax.experimental.pallas import tpu_sc as plsc

NUM_PROMPTS = 1000
SEQ_LEN = 77
TEXT_DIM = 4096
BATCH = 256

ROWS_PER_SC = BATCH // 2
NBUF = 4

_mesh = plsc.VectorSubcoreMesh(core_axis_name="c", subcore_axis_name="s")


@functools.partial(
    pl.kernel,
    mesh=_mesh,
    out_type=jax.ShapeDtypeStruct((BATCH, SEQ_LEN, TEXT_DIM), jnp.float32),
    compiler_params=pltpu.CompilerParams(
        needs_layout_passes=False, skip_device_barrier=True
    ),
    scratch_types=[
        pltpu.VMEM((BATCH,), jnp.int32),
        pltpu.VMEM_SHARED((NBUF, SEQ_LEN, TEXT_DIM), jnp.float32),
        pltpu.SemaphoreType.DMA,
        pltpu.SemaphoreType.DMA,
        pltpu.SemaphoreType.DMA,
        pltpu.SemaphoreType.DMA,
        pltpu.SemaphoreType.DMA,
    ],
)
def _sc_gather(table, idx_hbm, out, idx_v, ring, gs, s0, s1, s2, s3):
    cid = lax.axis_index("c")
    sid = lax.axis_index("s")
    base = cid * ROWS_PER_SC

    @pl.when(sid == 0)
    def _worker():
        pltpu.sync_copy(idx_hbm, idx_v)
        vecs = [idx_v[pl.ds(base + 16 * k, 16)] for k in range(ROWS_PER_SC // 16)]
        ids = [vecs[i // 16][i % 16] for i in range(ROWS_PER_SC)]
        sems = [s0, s1, s2, s3]

        def src(i):
            return table.at[pl.ds(ids[i], 1), :, :]

        def buf(i):
            return ring.at[pl.ds(i % NBUF, 1)]

        def dst(i):
            return out.at[pl.ds(base + i, 1), :, :]

        def start_gather(i):
            if i >= NBUF:
                pltpu.make_async_copy(
                    buf(i - NBUF), dst(i - NBUF), sems[i % NBUF]
                ).wait()
            pltpu.async_copy(src(i), buf(i), gs)

        start_gather(0)
        start_gather(1)
        for i in range(ROWS_PER_SC):
            if i + 2 < ROWS_PER_SC:
                start_gather(i + 2)
            pltpu.make_async_copy(src(i), buf(i), gs).wait()
            pltpu.async_copy(buf(i), dst(i), sems[i % NBUF])

        for i in range(ROWS_PER_SC - NBUF, ROWS_PER_SC):
            pltpu.make_async_copy(buf(i), dst(i), sems[i % NBUF]).wait()


def kernel(prompt_idx, embeddings):
    return _sc_gather(embeddings, prompt_idx.astype(jnp.int32))


# speedup vs baseline: 1.0070x; 1.0070x over previous
import functools

import jax
import jax.numpy as jnp
from jax import lax
from jax.experimental import pallas as pl
from jax.experimental.pallas import tpu as pltpu
from jax.experimental.pallas import tpu_sc as plsc

NUM_PROMPTS = 1000
SEQ_LEN = 77
TEXT_DIM = 4096
BATCH = 256

ROWS_PER_SC = BATCH // 2
WORKERS = 2
ROWS_PER_W = ROWS_PER_SC // WORKERS
NBUF = 3

_mesh = plsc.VectorSubcoreMesh(core_axis_name="c", subcore_axis_name="s")


@functools.partial(
    pl.kernel,
    mesh=_mesh,
    out_type=jax.ShapeDtypeStruct((BATCH, SEQ_LEN, TEXT_DIM), jnp.float32),
    compiler_params=pltpu.CompilerParams(needs_layout_passes=False),
    scratch_types=[
        pltpu.VMEM((BATCH,), jnp.int32),
        pltpu.VMEM_SHARED((WORKERS * NBUF, SEQ_LEN, TEXT_DIM), jnp.float32),
        pltpu.SemaphoreType.DMA,
        pltpu.SemaphoreType.DMA,
        pltpu.SemaphoreType.DMA,
    ],
)
def _sc_gather(table, idx_hbm, out, idx_v, ring, gs, p0, p1):
    cid = lax.axis_index("c")
    sid = lax.axis_index("s")

    def worker(w, psem):
        base = cid * ROWS_PER_SC + w * ROWS_PER_W
        pltpu.sync_copy(idx_hbm, idx_v)
        vecs = [idx_v[pl.ds(base + 16 * k, 16)] for k in range(ROWS_PER_W // 16)]
        ids = [vecs[i // 16][i % 16] for i in range(ROWS_PER_W)]

        def src(i):
            return table.at[pl.ds(ids[i], 1), :, :]

        def buf(i):
            return ring.at[pl.ds(w * NBUF + i % NBUF, 1)]

        def dst(i):
            return out.at[pl.ds(base + i, 1), :, :]

        def start_gather(i):
            if i >= NBUF:
                pltpu.make_async_copy(buf(i - NBUF), dst(i - NBUF), psem).wait()
            pltpu.async_copy(src(i), buf(i), gs)

        start_gather(0)
        start_gather(1)
        for i in range(ROWS_PER_W):
            if i + 2 < ROWS_PER_W:
                start_gather(i + 2)
            pltpu.make_async_copy(src(i), buf(i), gs).wait()
            pltpu.async_copy(buf(i), dst(i), psem)

        for i in range(ROWS_PER_W - NBUF, ROWS_PER_W):
            pltpu.make_async_copy(buf(i), dst(i), psem).wait()

    @pl.when(sid == 0)
    def _w0():
        worker(0, p0)

    @pl.when(sid == 1)
    def _w1():
        worker(1, p1)


def kernel(prompt_idx, embeddings):
    return _sc_gather(embeddings, prompt_idx.astype(jnp.int32))
